# baseline (speedup 1.0000x reference)
import functools

import jax
import jax.numpy as jnp
from jax import lax
from jax.experimental import pallas as pl
from jax.experimental.pallas import tpu as pltpu
from jax.experimental.pallas import tpu_sc as plsc

_NW = 32


def _make_sc_enqueue(B, D, K):
    rows_w = B // _NW
    mesh = plsc.VectorSubcoreMesh(core_axis_name="c", subcore_axis_name="s")

    @functools.partial(
        pl.kernel,
        mesh=mesh,
        scratch_types=[
            pltpu.VMEM((rows_w, D), jnp.float32),
            pltpu.VMEM((rows_w,), jnp.int32),
            pltpu.SemaphoreType.DMA,
            pltpu.SemaphoreType.DMA,
        ],
    )
    def k(feats_hbm, labels_hbm, fbank_ref, lbank_ref, fbuf, lbuf, sem0, sem1):
        wid = lax.axis_index("s") * 2 + lax.axis_index("c")
        lo = wid * rows_w
        cf = pltpu.async_copy(feats_hbm.at[pl.ds(lo, rows_w)], fbuf, sem0)
        cl = pltpu.async_copy(labels_hbm.at[pl.ds(lo, rows_w)], lbuf, sem1)
        cf.wait()
        cl.wait()
        of = pltpu.async_copy(fbuf, fbank_ref.at[pl.ds(lo, rows_w)], sem0)
        ol = pltpu.async_copy(lbuf, lbank_ref.at[pl.ds(lo, rows_w)], sem1)
        of.wait()
        ol.wait()

    return k


def kernel(feats, labels, features, labels_buf):
    B, D = feats.shape
    K = features.shape[0]

    fbank = jax.new_ref(features)
    lbank = jax.new_ref(labels_buf)
    _make_sc_enqueue(B, D, K)(feats, labels, fbank, lbank)
    out_f = fbank[...]
    out_l = lbank[...]

    new_ptr = jnp.full((1,), B % K, dtype=jnp.int32)
    return (out_f, out_l, new_ptr)
